# Initial kernel scaffold; baseline (speedup 1.0000x reference)
import jax
import jax.numpy as jnp
from jax import lax
from jax.experimental import pallas as pl
from jax.experimental.pallas import tpu as pltpu
from jax.experimental.pallas import tpu_sc as plsc

N = 10000
E = 320000
D = 128
H1 = 8
H2 = 64
PAD = 16

NC = 2
NS = 16
NW = NC * NS
EPW = E // NW
SUB = 80
NSUB = 5
CHUNK = SUB * NSUB
NCHUNK = EPW // CHUNK
RPT = N // NS


def _leaky(x):
    return jnp.where(x >= 0, x, 0.2 * x)


def _dense1_body(x_ref, w1_ref, a1_ref, zpad_ref, sd_ref, ss_ref, mt_ref):
    z = jnp.dot(x_ref[...], w1_ref[...], preferred_element_type=jnp.float32)
    a = a1_ref[...]
    sd = jnp.dot(z, a[0:H1, :], preferred_element_type=jnp.float32)
    ss = jnp.dot(z, a[H1:2 * H1, :], preferred_element_type=jnp.float32)
    zpad_ref[:, 0:H1] = z
    zpad_ref[:, H1:H1 + 1] = jnp.ones((N, 1), jnp.float32)
    zpad_ref[:, H1 + 1:PAD] = jnp.zeros((N, PAD - H1 - 1), jnp.float32)
    sd_ref[...] = sd
    ss_ref[...] = ss
    mt_ref[...] = _leaky(sd + ss)


_dense1 = pl.pallas_call(
    _dense1_body,
    out_shape=[
        jax.ShapeDtypeStruct((N, PAD), jnp.float32),
        jax.ShapeDtypeStruct((N, 1), jnp.float32),
        jax.ShapeDtypeStruct((N, 1), jnp.float32),
        jax.ShapeDtypeStruct((N, 1), jnp.float32),
    ],
)


def _dense2_body(acc_ref, zpad_ref, w2_ref, a2_ref,
                 hpad_ref, sd_ref, ss_ref, mt_ref):
    full = acc_ref[0] + acc_ref[1] + zpad_ref[...]
    h = full[:, 0:H1] / (full[:, H1:H1 + 1] + 1e-16)
    h = jnp.where(h > 0, h, jnp.expm1(h))
    z2 = jnp.dot(h, w2_ref[...], preferred_element_type=jnp.float32)
    a = a2_ref[...]
    sd = jnp.dot(z2, a[0:H2, :], preferred_element_type=jnp.float32)
    ss = jnp.dot(z2, a[H2:2 * H2, :], preferred_element_type=jnp.float32)
    hpad_ref[:, 0:H1] = h
    hpad_ref[:, H1:H1 + 1] = jnp.ones((N, 1), jnp.float32)
    hpad_ref[:, H1 + 1:PAD] = jnp.zeros((N, PAD - H1 - 1), jnp.float32)
    sd_ref[...] = sd
    ss_ref[...] = ss
    mt_ref[...] = _leaky(sd + ss)


_dense2 = pl.pallas_call(
    _dense2_body,
    out_shape=[
        jax.ShapeDtypeStruct((N, PAD), jnp.float32),
        jax.ShapeDtypeStruct((N, 1), jnp.float32),
        jax.ShapeDtypeStruct((N, 1), jnp.float32),
        jax.ShapeDtypeStruct((N, 1), jnp.float32),
    ],
)


def _dense3_body(acc_ref, hpad_ref, w2_ref, out_ref):
    full = acc_ref[0] + acc_ref[1] + hpad_ref[...]
    hbar = full[:, 0:H1] / (full[:, H1:H1 + 1] + 1e-16)
    o = jnp.dot(hbar, w2_ref[...], preferred_element_type=jnp.float32)
    m = jnp.max(o, axis=1, keepdims=True)
    lse = m + jnp.log(jnp.sum(jnp.exp(o - m), axis=1, keepdims=True))
    out_ref[...] = o - lse


_dense3 = pl.pallas_call(
    _dense3_body,
    out_shape=jax.ShapeDtypeStruct((N, H2), jnp.float32),
)


def _edge_body(src_hbm, dst_hbm, sd_hbm, ss_hbm, mt_hbm, zpad_hbm, out_hbm,
               sd_v, ss_v, mt_v, srci_v, dsti_v, ex_v, rows_v, zbuf_v,
               acc_sh, sem):
    c = lax.axis_index("c")
    s = lax.axis_index("s")
    wid = s * NC + c

    pltpu.sync_copy(sd_hbm, sd_v)
    pltpu.sync_copy(ss_hbm, ss_v)
    pltpu.sync_copy(mt_hbm, mt_v)

    def _zero(i, carry):
        zbuf_v[i, :] = jnp.zeros((16,), jnp.float32)
        return carry

    lax.fori_loop(0, RPT, _zero, 0)
    pltpu.sync_copy(zbuf_v, acc_sh.at[pl.ds(s * RPT, RPT), :])
    plsc.subcore_barrier()

    row0 = wid * (EPW // SUB)

    def _chunk(t, carry):
        r = row0 + t * NSUB
        pltpu.sync_copy(src_hbm.at[pl.ds(r, NSUB), :], srci_v)
        pltpu.sync_copy(dst_hbm.at[pl.ds(r, NSUB), :], dsti_v)

        for j in range(NSUB):
            for k in range(SUB // 16):
                di = dsti_v[j, pl.ds(k * 16, 16)]
                si = srci_v[j, pl.ds(k * 16, 16)]
                g1 = plsc.load_gather(sd_v, [di])
                g2 = plsc.load_gather(ss_v, [si])
                g3 = plsc.load_gather(mt_v, [di])
                e = _leaky(g1 + g2)
                ex_v[pl.ds(j * SUB + k * 16, 16)] = jnp.exp(e - g3)

        cps = [
            pltpu.async_copy(zpad_hbm.at[srci_v.at[j]],
                             rows_v.at[pl.ds(j * SUB, SUB), :], sem)
            for j in range(NSUB)
        ]
        for cp in cps:
            cp.wait()

        def _scale(b, carry2):
            w = plsc.load_gather(ex_v, [jnp.full((16,), b, jnp.int32)])
            rows_v[b, :] = rows_v[b, :] * w
            return carry2

        lax.fori_loop(0, CHUNK, _scale, 0)

        for j in range(NSUB):
            pltpu.sync_copy(rows_v.at[pl.ds(j * SUB, SUB), :],
                            acc_sh.at[dsti_v.at[j]], add=True)
        return carry

    lax.fori_loop(0, NCHUNK, _chunk, 0)
    plsc.subcore_barrier()

    pltpu.sync_copy(acc_sh.at[pl.ds(s * RPT, RPT), :],
                    out_hbm.at[c, pl.ds(s * RPT, RPT), :])


_edge = pl.kernel(
    _edge_body,
    out_type=jax.ShapeDtypeStruct((NC, N, PAD), jnp.float32),
    mesh=plsc.VectorSubcoreMesh(core_axis_name="c", subcore_axis_name="s"),
    scratch_types=[
        pltpu.VMEM((N,), jnp.float32),
        pltpu.VMEM((N,), jnp.float32),
        pltpu.VMEM((N,), jnp.float32),
        pltpu.VMEM((NSUB, SUB), jnp.int32),
        pltpu.VMEM((NSUB, SUB), jnp.int32),
        pltpu.VMEM((CHUNK,), jnp.float32),
        pltpu.VMEM((CHUNK, PAD), jnp.float32),
        pltpu.VMEM((RPT, PAD), jnp.float32),
        pltpu.VMEM_SHARED((N, PAD), jnp.float32),
        pltpu.SemaphoreType.DMA,
    ],
)


@jax.jit
def kernel(x, edge_index, W1, a1, W2, a2):
    ei = edge_index.astype(jnp.int32)
    src = ei[0].reshape(E // SUB, SUB)
    dst = ei[1].reshape(E // SUB, SUB)

    zpad1, sd1, ss1, mt1 = _dense1(x, W1, a1)
    acc1 = _edge(src, dst, sd1.reshape(N), ss1.reshape(N), mt1.reshape(N),
                 zpad1)
    hpad, sd2, ss2, mt2 = _dense2(acc1, zpad1, W2, a2)
    acc2 = _edge(src, dst, sd2.reshape(N), ss2.reshape(N), mt2.reshape(N),
                 hpad)
    return _dense3(acc2, hpad, W2)


# speedup vs baseline: 35.9776x; 35.9776x over previous
import jax
import jax.numpy as jnp
from jax import lax
from jax.experimental import pallas as pl
from jax.experimental.pallas import tpu as pltpu
from jax.experimental.pallas import tpu_sc as plsc

N = 10000
E = 320000
D = 128
H1 = 8
H2 = 64
PAD = 16

NC = 2
NS = 16
NW = NC * NS
EPW = E // NW
SUB = 80
NSUB = 5
CHUNK = SUB * NSUB
NCHUNK = EPW // CHUNK
RPT = 624
RPT_LAST = N - (NS - 1) * RPT


def _leaky(x):
    return jnp.where(x >= 0, x, 0.2 * x)


def _dense1_body(x_ref, w1_ref, a1_ref, zpad_ref, sd_ref, ss_ref, mt_ref):
    z = jnp.dot(x_ref[...], w1_ref[...], preferred_element_type=jnp.float32)
    a = a1_ref[...]
    sd = jnp.dot(z, a[0:H1, :], preferred_element_type=jnp.float32)
    ss = jnp.dot(z, a[H1:2 * H1, :], preferred_element_type=jnp.float32)
    zpad_ref[:, 0:H1] = z
    zpad_ref[:, H1:H1 + 1] = jnp.ones((N, 1), jnp.float32)
    zpad_ref[:, H1 + 1:PAD] = jnp.zeros((N, PAD - H1 - 1), jnp.float32)
    sd_ref[...] = sd
    ss_ref[...] = ss
    mt_ref[...] = _leaky(sd + ss)


_dense1 = pl.pallas_call(
    _dense1_body,
    out_shape=[
        jax.ShapeDtypeStruct((N, PAD), jnp.float32),
        jax.ShapeDtypeStruct((N, 1), jnp.float32),
        jax.ShapeDtypeStruct((N, 1), jnp.float32),
        jax.ShapeDtypeStruct((N, 1), jnp.float32),
    ],
)


def _dense2_body(acc_ref, zpad_ref, w2_ref, a2_ref,
                 hpad_ref, sd_ref, ss_ref, mt_ref):
    full = acc_ref[0] + acc_ref[1] + zpad_ref[...]
    h = full[:, 0:H1] / (full[:, H1:H1 + 1] + 1e-16)
    h = jnp.where(h > 0, h, jnp.exp(h) - 1.0)
    z2 = jnp.dot(h, w2_ref[...], preferred_element_type=jnp.float32)
    a = a2_ref[...]
    sd = jnp.dot(z2, a[0:H2, :], preferred_element_type=jnp.float32)
    ss = jnp.dot(z2, a[H2:2 * H2, :], preferred_element_type=jnp.float32)
    hpad_ref[:, 0:H1] = h
    hpad_ref[:, H1:H1 + 1] = jnp.ones((N, 1), jnp.float32)
    hpad_ref[:, H1 + 1:PAD] = jnp.zeros((N, PAD - H1 - 1), jnp.float32)
    sd_ref[...] = sd
    ss_ref[...] = ss
    mt_ref[...] = _leaky(sd + ss)


_dense2 = pl.pallas_call(
    _dense2_body,
    out_shape=[
        jax.ShapeDtypeStruct((N, PAD), jnp.float32),
        jax.ShapeDtypeStruct((N, 1), jnp.float32),
        jax.ShapeDtypeStruct((N, 1), jnp.float32),
        jax.ShapeDtypeStruct((N, 1), jnp.float32),
    ],
)


def _dense3_body(acc_ref, hpad_ref, w2_ref, out_ref):
    full = acc_ref[0] + acc_ref[1] + hpad_ref[...]
    hbar = full[:, 0:H1] / (full[:, H1:H1 + 1] + 1e-16)
    o = jnp.dot(hbar, w2_ref[...], preferred_element_type=jnp.float32)
    m = jnp.max(o, axis=1, keepdims=True)
    lse = m + jnp.log(jnp.sum(jnp.exp(o - m), axis=1, keepdims=True))
    out_ref[...] = o - lse


_dense3 = pl.pallas_call(
    _dense3_body,
    out_shape=jax.ShapeDtypeStruct((N, H2), jnp.float32),
)


def _edge_body(src_hbm, dst_hbm, sd_hbm, ss_hbm, mt_hbm, zpad_hbm, out_hbm,
               sd_v, ss_v, mt_v, srci_v, dsti_v, ex_v, rows_v, zbuf_v,
               acc_sh, sem):
    c = lax.axis_index("c")
    s = lax.axis_index("s")
    wid = s * NC + c

    pltpu.sync_copy(sd_hbm, sd_v)
    pltpu.sync_copy(ss_hbm, ss_v)
    pltpu.sync_copy(mt_hbm, mt_v)

    def _zero(i, carry):
        zbuf_v[i, :] = jnp.zeros((16,), jnp.float32)
        return carry

    lax.fori_loop(0, RPT_LAST, _zero, 0)
    nrow = jnp.where(s == NS - 1, RPT_LAST, RPT)
    pltpu.sync_copy(zbuf_v.at[pl.ds(0, nrow), :],
                    acc_sh.at[pl.ds(s * RPT, nrow), :])
    plsc.subcore_barrier()

    def _chunk(t, carry):
        cid = wid * NCHUNK + t
        pltpu.sync_copy(src_hbm.at[cid], srci_v)
        pltpu.sync_copy(dst_hbm.at[cid], dsti_v)

        for j in range(NSUB):
            for k in range(SUB // 16):
                di = dsti_v[j, pl.ds(k * 16, 16)]
                si = srci_v[j, pl.ds(k * 16, 16)]
                g1 = plsc.load_gather(sd_v, [di])
                g2 = plsc.load_gather(ss_v, [si])
                g3 = plsc.load_gather(mt_v, [di])
                e = _leaky(g1 + g2)
                ex_v[pl.ds(j * SUB + k * 16, 16)] = jnp.exp(e - g3)

        cps = [
            pltpu.async_copy(zpad_hbm.at[srci_v.at[j]],
                             rows_v.at[pl.ds(j * SUB, SUB), :], sem)
            for j in range(NSUB)
        ]
        for cp in cps:
            cp.wait()

        def _scale(b, carry2):
            w = plsc.load_gather(ex_v, [jnp.full((16,), b, jnp.int32)])
            rows_v[b, :] = rows_v[b, :] * w
            return carry2

        lax.fori_loop(0, CHUNK, _scale, 0)

        for j in range(NSUB):
            pltpu.sync_copy(rows_v.at[pl.ds(j * SUB, SUB), :],
                            acc_sh.at[dsti_v.at[j]], add=True)
        return carry

    lax.fori_loop(0, NCHUNK, _chunk, 0)
    plsc.subcore_barrier()

    pltpu.sync_copy(acc_sh.at[pl.ds(s * RPT, nrow), :],
                    out_hbm.at[c, pl.ds(s * RPT, nrow), :])


_edge = pl.kernel(
    _edge_body,
    out_type=jax.ShapeDtypeStruct((NC, N, PAD), jnp.float32),
    mesh=plsc.VectorSubcoreMesh(core_axis_name="c", subcore_axis_name="s",
                                num_cores=NC, num_subcores=NS),
    scratch_types=[
        pltpu.VMEM((N,), jnp.float32),
        pltpu.VMEM((N,), jnp.float32),
        pltpu.VMEM((N,), jnp.float32),
        pltpu.VMEM((NSUB, SUB), jnp.int32),
        pltpu.VMEM((NSUB, SUB), jnp.int32),
        pltpu.VMEM((CHUNK,), jnp.float32),
        pltpu.VMEM((CHUNK, PAD), jnp.float32),
        pltpu.VMEM((RPT_LAST, PAD), jnp.float32),
        pltpu.VMEM_SHARED((N, PAD), jnp.float32),
        pltpu.SemaphoreType.DMA,
    ],
    compiler_params=pltpu.CompilerParams(needs_layout_passes=False,
                                         use_tc_tiling_on_sc=False),
)


@jax.jit
def kernel(x, edge_index, W1, a1, W2, a2):
    ei = edge_index.astype(jnp.int32)
    src = ei[0].reshape(NW * NCHUNK, NSUB, SUB)
    dst = ei[1].reshape(NW * NCHUNK, NSUB, SUB)

    zpad1, sd1, ss1, mt1 = _dense1(x, W1, a1)
    acc1 = _edge(src, dst, sd1.reshape(N), ss1.reshape(N), mt1.reshape(N),
                 zpad1)
    hpad, sd2, ss2, mt2 = _dense2(acc1, zpad1, W2, a2)
    acc2 = _edge(src, dst, sd2.reshape(N), ss2.reshape(N), mt2.reshape(N),
                 hpad)
    return _dense3(acc2, hpad, W2)


# speedup vs baseline: 51.6511x; 1.4356x over previous
import jax
import jax.numpy as jnp
from jax import lax
from jax.experimental import pallas as pl
from jax.experimental.pallas import tpu as pltpu
from jax.experimental.pallas import tpu_sc as plsc

N = 10000
NP = 10240
E = 320000
EP = 327680
D = 128
H1 = 8
H2 = 64
PAD = 16

NC = 2
NS = 16
NW = NC * NS
SUB = 128
NSUB = 16
CHUNK = SUB * NSUB
NCHUNK = EP // (NW * CHUNK)
RPT = NP // NS


def _leaky(x):
    return jnp.where(x >= 0, x, 0.2 * x)


def _dense1_body(x_ref, w1_ref, a1_ref, zpad_ref, sd_ref, ss_ref, mt_ref):
    z = jnp.dot(x_ref[...], w1_ref[...], preferred_element_type=jnp.float32)
    a = a1_ref[...]
    sd = jnp.dot(z, a[0:H1, :], preferred_element_type=jnp.float32)
    ss = jnp.dot(z, a[H1:2 * H1, :], preferred_element_type=jnp.float32)
    zpad_ref[0:N, 0:H1] = z
    zpad_ref[0:N, H1:H1 + 1] = jnp.ones((N, 1), jnp.float32)
    zpad_ref[0:N, H1 + 1:PAD] = jnp.zeros((N, PAD - H1 - 1), jnp.float32)
    zpad_ref[N:NP, :] = jnp.zeros((NP - N, PAD), jnp.float32)
    sd_ref[0:N] = sd
    ss_ref[0:N] = ss
    mt_ref[0:N] = _leaky(sd + ss)
    sd_ref[N:NP] = jnp.zeros((NP - N, 1), jnp.float32)
    ss_ref[N:NP] = jnp.zeros((NP - N, 1), jnp.float32)
    mt_ref[N:NP] = jnp.zeros((NP - N, 1), jnp.float32)


_dense1 = pl.pallas_call(
    _dense1_body,
    out_shape=[
        jax.ShapeDtypeStruct((NP, PAD), jnp.float32),
        jax.ShapeDtypeStruct((NP, 1), jnp.float32),
        jax.ShapeDtypeStruct((NP, 1), jnp.float32),
        jax.ShapeDtypeStruct((NP, 1), jnp.float32),
    ],
)


def _dense2_body(acc_ref, zpad_ref, w2_ref, a2_ref,
                 hpad_ref, sd_ref, ss_ref, mt_ref):
    full = acc_ref[0] + acc_ref[1] + zpad_ref[...]
    h = full[:, 0:H1] / (full[:, H1:H1 + 1] + 1e-16)
    h = jnp.where(h > 0, h, jnp.exp(h) - 1.0)
    z2 = jnp.dot(h, w2_ref[...], preferred_element_type=jnp.float32)
    a = a2_ref[...]
    sd = jnp.dot(z2, a[0:H2, :], preferred_element_type=jnp.float32)
    ss = jnp.dot(z2, a[H2:2 * H2, :], preferred_element_type=jnp.float32)
    hpad_ref[:, 0:H1] = h
    hpad_ref[:, H1:H1 + 1] = jnp.ones((NP, 1), jnp.float32)
    hpad_ref[:, H1 + 1:PAD] = jnp.zeros((NP, PAD - H1 - 1), jnp.float32)
    sd_ref[...] = sd
    ss_ref[...] = ss
    mt_ref[...] = _leaky(sd + ss)


_dense2 = pl.pallas_call(
    _dense2_body,
    out_shape=[
        jax.ShapeDtypeStruct((NP, PAD), jnp.float32),
        jax.ShapeDtypeStruct((NP, 1), jnp.float32),
        jax.ShapeDtypeStruct((NP, 1), jnp.float32),
        jax.ShapeDtypeStruct((NP, 1), jnp.float32),
    ],
)


def _dense3_body(acc_ref, hpad_ref, w2_ref, out_ref):
    full = acc_ref[0] + acc_ref[1] + hpad_ref[...]
    hbar = full[0:N, 0:H1] / (full[0:N, H1:H1 + 1] + 1e-16)
    o = jnp.dot(hbar, w2_ref[...], preferred_element_type=jnp.float32)
    m = jnp.max(o, axis=1, keepdims=True)
    lse = m + jnp.log(jnp.sum(jnp.exp(o - m), axis=1, keepdims=True))
    out_ref[...] = o - lse


_dense3 = pl.pallas_call(
    _dense3_body,
    out_shape=jax.ShapeDtypeStruct((N, H2), jnp.float32),
)


def _edge_body(src_hbm, dst_hbm, sd_hbm, ss_hbm, mt_hbm, zpad_hbm, out_hbm,
               sd_v, ss_v, mt_v, srci_v, dsti_v, ex_v, rows_v,
               acc_sh, isem, gsem, ssem):
    c = lax.axis_index("c")
    s = lax.axis_index("s")
    wid = s * NC + c

    pltpu.sync_copy(sd_hbm, sd_v)
    pltpu.sync_copy(ss_hbm, ss_v)
    pltpu.sync_copy(mt_hbm, mt_v)

    @plsc.parallel_loop(0, RPT, step=1, unroll=8)
    def _zero(i):
        rows_v[i, :] = jnp.zeros((16,), jnp.float32)

    pltpu.sync_copy(rows_v.at[pl.ds(0, RPT), :],
                    acc_sh.at[pl.ds(s * RPT, RPT), :])
    plsc.subcore_barrier()

    def _chunk(t, carry):
        cid = wid * NCHUNK + t
        cp_s = pltpu.async_copy(src_hbm.at[cid], srci_v, isem)
        cp_d = pltpu.async_copy(dst_hbm.at[cid], dsti_v, isem)
        cp_s.wait()
        cp_d.wait()

        gathers = [
            pltpu.async_copy(zpad_hbm.at[srci_v.at[j]],
                             rows_v.at[pl.ds(j * SUB, SUB), :], gsem)
            for j in range(NSUB)
        ]

        @plsc.parallel_loop(0, NSUB, step=1, unroll=2)
        def _exw(j):
            for k in range(SUB // 16):
                di = dsti_v[j, pl.ds(k * 16, 16)]
                si = srci_v[j, pl.ds(k * 16, 16)]
                g1 = plsc.load_gather(sd_v, [di])
                g2 = plsc.load_gather(ss_v, [si])
                g3 = plsc.load_gather(mt_v, [di])
                e = _leaky(g1 + g2)
                ex_v[pl.ds(j * SUB + k * 16, 16)] = jnp.exp(e - g3)

        for cp in gathers:
            cp.wait()

        @plsc.parallel_loop(0, CHUNK, step=1, unroll=8)
        def _scale(b):
            w = plsc.load_gather(ex_v, [jnp.full((16,), b, jnp.int32)])
            rows_v[b, :] = rows_v[b, :] * w

        scatters = [
            pltpu.async_copy(rows_v.at[pl.ds(j * SUB, SUB), :],
                             acc_sh.at[dsti_v.at[j]], ssem, add=True)
            for j in range(NSUB)
        ]
        for cp in scatters:
            cp.wait()
        return carry

    lax.fori_loop(0, NCHUNK, _chunk, 0)
    plsc.subcore_barrier()

    pltpu.sync_copy(acc_sh.at[pl.ds(s * RPT, RPT), :],
                    out_hbm.at[c, pl.ds(s * RPT, RPT), :])


_edge = pl.kernel(
    _edge_body,
    out_type=jax.ShapeDtypeStruct((NC, NP, PAD), jnp.float32),
    mesh=plsc.VectorSubcoreMesh(core_axis_name="c", subcore_axis_name="s",
                                num_cores=NC, num_subcores=NS),
    scratch_types=[
        pltpu.VMEM((NP,), jnp.float32),
        pltpu.VMEM((NP,), jnp.float32),
        pltpu.VMEM((NP,), jnp.float32),
        pltpu.VMEM((NSUB, SUB), jnp.int32),
        pltpu.VMEM((NSUB, SUB), jnp.int32),
        pltpu.VMEM((CHUNK,), jnp.float32),
        pltpu.VMEM((CHUNK, PAD), jnp.float32),
        pltpu.VMEM_SHARED((NP, PAD), jnp.float32),
        pltpu.SemaphoreType.DMA,
        pltpu.SemaphoreType.DMA,
        pltpu.SemaphoreType.DMA,
    ],
    compiler_params=pltpu.CompilerParams(needs_layout_passes=False,
                                         use_tc_tiling_on_sc=False),
)


@jax.jit
def kernel(x, edge_index, W1, a1, W2, a2):
    ei = edge_index.astype(jnp.int32)
    fill = jnp.full((EP - E,), N, jnp.int32)
    src = jnp.concatenate([ei[0], fill]).reshape(NW * NCHUNK, NSUB, SUB)
    dst = jnp.concatenate([ei[1], fill]).reshape(NW * NCHUNK, NSUB, SUB)

    zpad1, sd1, ss1, mt1 = _dense1(x, W1, a1)
    acc1 = _edge(src, dst, sd1.reshape(NP), ss1.reshape(NP), mt1.reshape(NP),
                 zpad1)
    hpad, sd2, ss2, mt2 = _dense2(acc1, zpad1, W2, a2)
    acc2 = _edge(src, dst, sd2.reshape(NP), ss2.reshape(NP), mt2.reshape(NP),
                 hpad)
    return _dense3(acc2, hpad, W2)


# speedup vs baseline: 55.4761x; 1.0741x over previous
import jax
import jax.numpy as jnp
from jax import lax
from jax.experimental import pallas as pl
from jax.experimental.pallas import tpu as pltpu
from jax.experimental.pallas import tpu_sc as plsc

N = 10000
NP = 10240
E = 320000
EP = 327680
D = 128
H1 = 8
H2 = 64
PAD = 16

NC = 2
NS = 16
NW = NC * NS
SUB = 128
NSUB = 8
CHUNK = SUB * NSUB
NCHUNK = EP // (NW * CHUNK)
RPT = NP // NS


def _leaky(x):
    return jnp.where(x >= 0, x, 0.2 * x)


def _dense1_body(x_ref, w1_ref, a1_ref, zpad_ref, sd_ref, ss_ref, mt_ref):
    z = jnp.dot(x_ref[...], w1_ref[...], preferred_element_type=jnp.float32)
    a = a1_ref[...]
    sd = jnp.dot(z, a[0:H1, :], preferred_element_type=jnp.float32)
    ss = jnp.dot(z, a[H1:2 * H1, :], preferred_element_type=jnp.float32)
    zpad_ref[0:N, 0:H1] = z
    zpad_ref[0:N, H1:H1 + 1] = jnp.ones((N, 1), jnp.float32)
    zpad_ref[0:N, H1 + 1:PAD] = jnp.zeros((N, PAD - H1 - 1), jnp.float32)
    zpad_ref[N:NP, :] = jnp.zeros((NP - N, PAD), jnp.float32)
    sd_ref[0:N] = sd
    ss_ref[0:N] = ss
    mt_ref[0:N] = _leaky(sd + ss)
    sd_ref[N:NP] = jnp.zeros((NP - N, 1), jnp.float32)
    ss_ref[N:NP] = jnp.zeros((NP - N, 1), jnp.float32)
    mt_ref[N:NP] = jnp.zeros((NP - N, 1), jnp.float32)


_dense1 = pl.pallas_call(
    _dense1_body,
    out_shape=[
        jax.ShapeDtypeStruct((NP, PAD), jnp.float32),
        jax.ShapeDtypeStruct((NP, 1), jnp.float32),
        jax.ShapeDtypeStruct((NP, 1), jnp.float32),
        jax.ShapeDtypeStruct((NP, 1), jnp.float32),
    ],
)


def _dense2_body(acc_ref, zpad_ref, w2_ref, a2_ref,
                 hpad_ref, sd_ref, ss_ref, mt_ref):
    full = acc_ref[0] + acc_ref[1] + zpad_ref[...]
    h = full[:, 0:H1] / (full[:, H1:H1 + 1] + 1e-16)
    h = jnp.where(h > 0, h, jnp.exp(h) - 1.0)
    z2 = jnp.dot(h, w2_ref[...], preferred_element_type=jnp.float32)
    a = a2_ref[...]
    sd = jnp.dot(z2, a[0:H2, :], preferred_element_type=jnp.float32)
    ss = jnp.dot(z2, a[H2:2 * H2, :], preferred_element_type=jnp.float32)
    hpad_ref[:, 0:H1] = h
    hpad_ref[:, H1:H1 + 1] = jnp.ones((NP, 1), jnp.float32)
    hpad_ref[:, H1 + 1:PAD] = jnp.zeros((NP, PAD - H1 - 1), jnp.float32)
    sd_ref[...] = sd
    ss_ref[...] = ss
    mt_ref[...] = _leaky(sd + ss)


_dense2 = pl.pallas_call(
    _dense2_body,
    out_shape=[
        jax.ShapeDtypeStruct((NP, PAD), jnp.float32),
        jax.ShapeDtypeStruct((NP, 1), jnp.float32),
        jax.ShapeDtypeStruct((NP, 1), jnp.float32),
        jax.ShapeDtypeStruct((NP, 1), jnp.float32),
    ],
)


def _dense3_body(acc_ref, hpad_ref, w2_ref, out_ref):
    full = acc_ref[0] + acc_ref[1] + hpad_ref[...]
    hbar = full[0:N, 0:H1] / (full[0:N, H1:H1 + 1] + 1e-16)
    o = jnp.dot(hbar, w2_ref[...], preferred_element_type=jnp.float32)
    m = jnp.max(o, axis=1, keepdims=True)
    lse = m + jnp.log(jnp.sum(jnp.exp(o - m), axis=1, keepdims=True))
    out_ref[...] = o - lse


_dense3 = pl.pallas_call(
    _dense3_body,
    out_shape=jax.ShapeDtypeStruct((N, H2), jnp.float32),
)


def _edge_body(src_hbm, dst_hbm, sd_hbm, ss_hbm, mt_hbm, zpad_hbm, out_hbm,
               sd_v, ss_v, mt_v, srci_a, dsti_a, ex_a, rows_a,
               srci_b, dsti_b, ex_b, rows_b,
               acc_sh, isem_a, isem_b, gsem_a, gsem_b, ssem_a, ssem_b):
    c = lax.axis_index("c")
    s = lax.axis_index("s")
    wid = s * NC + c

    tbl = [pltpu.async_copy(sd_hbm, sd_v, isem_a),
           pltpu.async_copy(ss_hbm, ss_v, isem_a),
           pltpu.async_copy(mt_hbm, mt_v, isem_a)]

    @plsc.parallel_loop(0, RPT // 2, step=1, unroll=8)
    def _zero(i):
        rows_a[i, :] = jnp.zeros((16,), jnp.float32)
        rows_b[i, :] = jnp.zeros((16,), jnp.float32)

    pltpu.sync_copy(rows_a.at[pl.ds(0, RPT // 2), :],
                    acc_sh.at[pl.ds(s * RPT, RPT // 2), :])
    pltpu.sync_copy(rows_b.at[pl.ds(0, RPT // 2), :],
                    acc_sh.at[pl.ds(s * RPT + RPT // 2, RPT // 2), :])
    for cp in tbl:
        cp.wait()
    plsc.subcore_barrier()

    def _ex_phase(srci_v, dsti_v, ex_v):
        @plsc.parallel_loop(0, NSUB, step=1, unroll=2)
        def _exw(j):
            for k in range(SUB // 16):
                di = dsti_v[j, pl.ds(k * 16, 16)]
                si = srci_v[j, pl.ds(k * 16, 16)]
                g1 = plsc.load_gather(sd_v, [di])
                g2 = plsc.load_gather(ss_v, [si])
                g3 = plsc.load_gather(mt_v, [di])
                e = _leaky(g1 + g2)
                ex_v[pl.ds(j * SUB + k * 16, 16)] = jnp.exp(e - g3)

    def _scale_phase(ex_v, rows_v):
        @plsc.parallel_loop(0, CHUNK, step=1, unroll=8)
        def _scale(b):
            w = plsc.load_gather(ex_v, [jnp.full((16,), b, jnp.int32)])
            rows_v[b, :] = rows_v[b, :] * w

    def _fire_gathers(srci_v, rows_v, gsem):
        return [
            pltpu.async_copy(zpad_hbm.at[srci_v.at[j]],
                             rows_v.at[pl.ds(j * SUB, SUB), :], gsem)
            for j in range(NSUB)
        ]

    def _fire_scatters(dsti_v, rows_v, ssem):
        return [
            pltpu.async_copy(rows_v.at[pl.ds(j * SUB, SUB), :],
                             acc_sh.at[dsti_v.at[j]], ssem, add=True)
            for j in range(NSUB)
        ]

    def _pair(g, carry):
        cid_a = wid * NCHUNK + 2 * g
        cid_b = cid_a + 1
        ids_a = [pltpu.async_copy(src_hbm.at[cid_a], srci_a, isem_a),
                 pltpu.async_copy(dst_hbm.at[cid_a], dsti_a, isem_a)]
        ids_b = [pltpu.async_copy(src_hbm.at[cid_b], srci_b, isem_b),
                 pltpu.async_copy(dst_hbm.at[cid_b], dsti_b, isem_b)]
        for cp in ids_a:
            cp.wait()
        gat_a = _fire_gathers(srci_a, rows_a, gsem_a)
        _ex_phase(srci_a, dsti_a, ex_a)
        for cp in ids_b:
            cp.wait()
        gat_b = _fire_gathers(srci_b, rows_b, gsem_b)
        for cp in gat_a:
            cp.wait()
        _scale_phase(ex_a, rows_a)
        sca_a = _fire_scatters(dsti_a, rows_a, ssem_a)
        _ex_phase(srci_b, dsti_b, ex_b)
        for cp in gat_b:
            cp.wait()
        _scale_phase(ex_b, rows_b)
        sca_b = _fire_scatters(dsti_b, rows_b, ssem_b)
        for cp in sca_a:
            cp.wait()
        for cp in sca_b:
            cp.wait()
        return carry

    lax.fori_loop(0, NCHUNK // 2, _pair, 0)
    plsc.subcore_barrier()

    pltpu.sync_copy(acc_sh.at[pl.ds(s * RPT, RPT), :],
                    out_hbm.at[c, pl.ds(s * RPT, RPT), :])


_edge = pl.kernel(
    _edge_body,
    out_type=jax.ShapeDtypeStruct((NC, NP, PAD), jnp.float32),
    mesh=plsc.VectorSubcoreMesh(core_axis_name="c", subcore_axis_name="s",
                                num_cores=NC, num_subcores=NS),
    scratch_types=[
        pltpu.VMEM((NP,), jnp.float32),
        pltpu.VMEM((NP,), jnp.float32),
        pltpu.VMEM((NP,), jnp.float32),
        pltpu.VMEM((NSUB, SUB), jnp.int32),
        pltpu.VMEM((NSUB, SUB), jnp.int32),
        pltpu.VMEM((CHUNK,), jnp.float32),
        pltpu.VMEM((CHUNK, PAD), jnp.float32),
        pltpu.VMEM((NSUB, SUB), jnp.int32),
        pltpu.VMEM((NSUB, SUB), jnp.int32),
        pltpu.VMEM((CHUNK,), jnp.float32),
        pltpu.VMEM((CHUNK, PAD), jnp.float32),
        pltpu.VMEM_SHARED((NP, PAD), jnp.float32),
        pltpu.SemaphoreType.DMA,
        pltpu.SemaphoreType.DMA,
        pltpu.SemaphoreType.DMA,
        pltpu.SemaphoreType.DMA,
        pltpu.SemaphoreType.DMA,
        pltpu.SemaphoreType.DMA,
    ],
    compiler_params=pltpu.CompilerParams(needs_layout_passes=False,
                                         use_tc_tiling_on_sc=False),
)


@jax.jit
def kernel(x, edge_index, W1, a1, W2, a2):
    ei = edge_index.astype(jnp.int32)
    fill = jnp.full((EP - E,), N, jnp.int32)
    src = jnp.concatenate([ei[0], fill]).reshape(NW * NCHUNK, NSUB, SUB)
    dst = jnp.concatenate([ei[1], fill]).reshape(NW * NCHUNK, NSUB, SUB)

    zpad1, sd1, ss1, mt1 = _dense1(x, W1, a1)
    acc1 = _edge(src, dst, sd1.reshape(NP), ss1.reshape(NP), mt1.reshape(NP),
                 zpad1)
    hpad, sd2, ss2, mt2 = _dense2(acc1, zpad1, W2, a2)
    acc2 = _edge(src, dst, sd2.reshape(NP), ss2.reshape(NP), mt2.reshape(NP),
                 hpad)
    return _dense3(acc2, hpad, W2)


# speedup vs baseline: 59.9652x; 1.0809x over previous
import jax
import jax.numpy as jnp
from jax import lax
from jax.experimental import pallas as pl
from jax.experimental.pallas import tpu as pltpu
from jax.experimental.pallas import tpu_sc as plsc

N = 10000
NP = 10240
E = 320000
EP = 327680
D = 128
H1 = 8
H2 = 64
PAD = 8

NC = 2
NS = 16
NW = NC * NS
SUB = 128
NSUB = 8
CHUNK = SUB * NSUB
NCHUNK = EP // (NW * CHUNK)
RPT = NP // NS


def _leaky(x):
    return jnp.where(x >= 0, x, 0.2 * x)


def _dense1_body(x_ref, w1_ref, a1_ref, zpad_ref, sd_ref, ss_ref, mt_ref):
    z = jnp.dot(x_ref[...], w1_ref[...], preferred_element_type=jnp.float32)
    a = a1_ref[...]
    sd = jnp.dot(z, a[0:H1, :], preferred_element_type=jnp.float32)
    ss = jnp.dot(z, a[H1:2 * H1, :], preferred_element_type=jnp.float32)
    zpad_ref[0:N, :] = z
    zpad_ref[N:NP, :] = jnp.zeros((NP - N, PAD), jnp.float32)
    sd_ref[0:N] = sd
    ss_ref[0:N] = ss
    mt_ref[0:N] = _leaky(sd + ss)
    sd_ref[N:NP] = jnp.zeros((NP - N, 1), jnp.float32)
    ss_ref[N:NP] = jnp.zeros((NP - N, 1), jnp.float32)
    mt_ref[N:NP] = jnp.zeros((NP - N, 1), jnp.float32)


_dense1 = pl.pallas_call(
    _dense1_body,
    out_shape=[
        jax.ShapeDtypeStruct((NP, PAD), jnp.float32),
        jax.ShapeDtypeStruct((NP, 1), jnp.float32),
        jax.ShapeDtypeStruct((NP, 1), jnp.float32),
        jax.ShapeDtypeStruct((NP, 1), jnp.float32),
    ],
)


def _dense2_body(acc_ref, dsum_ref, zpad_ref, w2_ref, a2_ref,
                 hpad_ref, sd_ref, ss_ref, mt_ref):
    numer = acc_ref[0] + acc_ref[1] + zpad_ref[...]
    d = jnp.sum(dsum_ref[0], axis=0) + jnp.sum(dsum_ref[1], axis=0) + 1.0
    h = numer / (jnp.expand_dims(d, 1) + 1e-16)
    h = jnp.where(h > 0, h, jnp.exp(h) - 1.0)
    z2 = jnp.dot(h, w2_ref[...], preferred_element_type=jnp.float32)
    a = a2_ref[...]
    sd = jnp.dot(z2, a[0:H2, :], preferred_element_type=jnp.float32)
    ss = jnp.dot(z2, a[H2:2 * H2, :], preferred_element_type=jnp.float32)
    hpad_ref[...] = h
    sd_ref[...] = sd
    ss_ref[...] = ss
    mt_ref[...] = _leaky(sd + ss)


_dense2 = pl.pallas_call(
    _dense2_body,
    out_shape=[
        jax.ShapeDtypeStruct((NP, PAD), jnp.float32),
        jax.ShapeDtypeStruct((NP, 1), jnp.float32),
        jax.ShapeDtypeStruct((NP, 1), jnp.float32),
        jax.ShapeDtypeStruct((NP, 1), jnp.float32),
    ],
)


def _dense3_body(acc_ref, dsum_ref, hpad_ref, w2_ref, out_ref):
    numer = acc_ref[0] + acc_ref[1] + hpad_ref[...]
    d = jnp.sum(dsum_ref[0], axis=0) + jnp.sum(dsum_ref[1], axis=0) + 1.0
    hbar = numer[0:N, :] / (jnp.expand_dims(d[0:N], 1) + 1e-16)
    o = jnp.dot(hbar, w2_ref[...], preferred_element_type=jnp.float32)
    m = jnp.max(o, axis=1, keepdims=True)
    lse = m + jnp.log(jnp.sum(jnp.exp(o - m), axis=1, keepdims=True))
    out_ref[...] = o - lse


_dense3 = pl.pallas_call(
    _dense3_body,
    out_shape=jax.ShapeDtypeStruct((N, H2), jnp.float32),
)


def _edge_body(src_hbm, dst_hbm, sd_hbm, ss_hbm, mt_hbm, zpad_hbm,
               out_hbm, dout_hbm,
               sd_v, ss_v, mt_v, den_v, srci_a, dsti_a, ex_a, rows_a,
               srci_b, dsti_b, ex_b, rows_b,
               acc_sh, isem_a, isem_b, gsem_a, gsem_b, ssem_a, ssem_b):
    c = lax.axis_index("c")
    s = lax.axis_index("s")
    wid = s * NC + c

    tbl = [pltpu.async_copy(sd_hbm, sd_v, isem_a),
           pltpu.async_copy(ss_hbm, ss_v, isem_a),
           pltpu.async_copy(mt_hbm, mt_v, isem_a)]

    @plsc.parallel_loop(0, NP // 16, step=1, unroll=8)
    def _zerod(i):
        den_v[pl.ds(i * 16, 16)] = jnp.zeros((16,), jnp.float32)

    lane0 = lax.iota(jnp.int32, 16)
    rh0 = lane0 // 8
    ch0 = lane0 % 8

    @plsc.parallel_loop(0, RPT // 2, step=1, unroll=8)
    def _zero(i):
        plsc.store_scatter(rows_a, [2 * i + rh0, ch0],
                           jnp.zeros((16,), jnp.float32))

    pltpu.sync_copy(rows_a.at[pl.ds(0, RPT), :],
                    acc_sh.at[pl.ds(s * RPT, RPT), :])
    for cp in tbl:
        cp.wait()
    plsc.subcore_barrier()

    def _ex_phase(srci_v, dsti_v, ex_v):
        @plsc.parallel_loop(0, NSUB, step=1, unroll=2)
        def _exw(j):
            for k in range(SUB // 16):
                di = dsti_v[j, pl.ds(k * 16, 16)]
                si = srci_v[j, pl.ds(k * 16, 16)]
                g1 = plsc.load_gather(sd_v, [di])
                g2 = plsc.load_gather(ss_v, [si])
                g3 = plsc.load_gather(mt_v, [di])
                e = _leaky(g1 + g2)
                w = jnp.exp(e - g3)
                ex_v[pl.ds(j * SUB + k * 16, 16)] = w
                plsc.addupdate_scatter(den_v, [di], w)

    rhalf = rh0
    chalf = ch0

    def _scale_phase(ex_v, rows_v):
        @plsc.parallel_loop(0, CHUNK // 2, step=1, unroll=8)
        def _scale(b):
            ridx = 2 * b + rhalf
            w = plsc.load_gather(ex_v, [ridx])
            r = plsc.load_gather(rows_v, [ridx, chalf])
            plsc.store_scatter(rows_v, [ridx, chalf], r * w)

    def _fire_gathers(srci_v, rows_v, gsem):
        return [
            pltpu.async_copy(zpad_hbm.at[srci_v.at[j]],
                             rows_v.at[pl.ds(j * SUB, SUB), :], gsem)
            for j in range(NSUB)
        ]

    def _fire_scatters(dsti_v, rows_v, ssem):
        return [
            pltpu.async_copy(rows_v.at[pl.ds(j * SUB, SUB), :],
                             acc_sh.at[dsti_v.at[j]], ssem, add=True)
            for j in range(NSUB)
        ]

    def _pair(g, carry):
        cid_a = wid * NCHUNK + 2 * g
        cid_b = cid_a + 1
        ids_a = [pltpu.async_copy(src_hbm.at[cid_a], srci_a, isem_a),
                 pltpu.async_copy(dst_hbm.at[cid_a], dsti_a, isem_a)]
        ids_b = [pltpu.async_copy(src_hbm.at[cid_b], srci_b, isem_b),
                 pltpu.async_copy(dst_hbm.at[cid_b], dsti_b, isem_b)]
        for cp in ids_a:
            cp.wait()
        gat_a = _fire_gathers(srci_a, rows_a, gsem_a)
        _ex_phase(srci_a, dsti_a, ex_a)
        for cp in ids_b:
            cp.wait()
        gat_b = _fire_gathers(srci_b, rows_b, gsem_b)
        for cp in gat_a:
            cp.wait()
        _scale_phase(ex_a, rows_a)
        sca_a = _fire_scatters(dsti_a, rows_a, ssem_a)
        _ex_phase(srci_b, dsti_b, ex_b)
        for cp in gat_b:
            cp.wait()
        _scale_phase(ex_b, rows_b)
        sca_b = _fire_scatters(dsti_b, rows_b, ssem_b)
        for cp in sca_a:
            cp.wait()
        for cp in sca_b:
            cp.wait()
        return carry

    lax.fori_loop(0, NCHUNK // 2, _pair, 0)
    plsc.subcore_barrier()

    pltpu.sync_copy(acc_sh.at[pl.ds(s * RPT, RPT), :],
                    out_hbm.at[c, pl.ds(s * RPT, RPT), :])
    pltpu.sync_copy(den_v, dout_hbm.at[c, s])


_edge = pl.kernel(
    _edge_body,
    out_type=(jax.ShapeDtypeStruct((NC, NP, PAD), jnp.float32),
              jax.ShapeDtypeStruct((NC, NS, NP), jnp.float32)),
    mesh=plsc.VectorSubcoreMesh(core_axis_name="c", subcore_axis_name="s",
                                num_cores=NC, num_subcores=NS),
    scratch_types=[
        pltpu.VMEM((NP,), jnp.float32),
        pltpu.VMEM((NP,), jnp.float32),
        pltpu.VMEM((NP,), jnp.float32),
        pltpu.VMEM((NP,), jnp.float32),
        pltpu.VMEM((NSUB, SUB), jnp.int32),
        pltpu.VMEM((NSUB, SUB), jnp.int32),
        pltpu.VMEM((CHUNK,), jnp.float32),
        pltpu.VMEM((CHUNK, PAD), jnp.float32),
        pltpu.VMEM((NSUB, SUB), jnp.int32),
        pltpu.VMEM((NSUB, SUB), jnp.int32),
        pltpu.VMEM((CHUNK,), jnp.float32),
        pltpu.VMEM((CHUNK, PAD), jnp.float32),
        pltpu.VMEM_SHARED((NP, PAD), jnp.float32),
        pltpu.SemaphoreType.DMA,
        pltpu.SemaphoreType.DMA,
        pltpu.SemaphoreType.DMA,
        pltpu.SemaphoreType.DMA,
        pltpu.SemaphoreType.DMA,
        pltpu.SemaphoreType.DMA,
    ],
    compiler_params=pltpu.CompilerParams(needs_layout_passes=False,
                                         use_tc_tiling_on_sc=False),
)


@jax.jit
def kernel(x, edge_index, W1, a1, W2, a2):
    ei = edge_index.astype(jnp.int32)
    fill = jnp.full((EP - E,), N, jnp.int32)
    src = jnp.concatenate([ei[0], fill]).reshape(NW * NCHUNK, NSUB, SUB)
    dst = jnp.concatenate([ei[1], fill]).reshape(NW * NCHUNK, NSUB, SUB)

    zpad1, sd1, ss1, mt1 = _dense1(x, W1, a1)
    acc1, dsum1 = _edge(src, dst, sd1.reshape(NP), ss1.reshape(NP),
                        mt1.reshape(NP), zpad1)
    hpad, sd2, ss2, mt2 = _dense2(acc1, dsum1, zpad1, W2, a2)
    acc2, dsum2 = _edge(src, dst, sd2.reshape(NP), ss2.reshape(NP),
                        mt2.reshape(NP), hpad)
    return _dense3(acc2, dsum2, hpad, W2)
